# baseline (device time: 11375 ns/iter reference)
import os

import jax
import jax.numpy as jnp
from jax import lax
from jax.experimental import pallas as pl
from jax.experimental.pallas import tpu as pltpu

N_DEV = 16
_NOCOMM = os.environ.get("KERNEL_NOCOMM") == "1"
_NOBAR = os.environ.get("KERNEL_NOBAR") == "1"
_AUTOBAR = os.environ.get("KERNEL_AUTOBAR") == "1"
_CREDIT = os.environ.get("KERNEL_CREDIT") == "1"


def kernel(x):
    m, n = x.shape
    B = 128
    G = m // B

    def body(x_hbm, out_hbm, xv_ref, stage_ref, tot_ref, comm_ref,
             send_sems, recv_sems, in_sem, out_sems, ready_sems):
        my_pos = lax.axis_index("i")

        if not _NOCOMM and not _AUTOBAR:
            barrier_sem = pltpu.get_barrier_semaphore()
            if _NOBAR or _CREDIT:
                pl.semaphore_signal(barrier_sem, inc=1)
            else:
                for j in range(N_DEV):
                    @pl.when(j != my_pos)
                    def _():
                        pl.semaphore_signal(
                            barrier_sem, inc=1,
                            device_id=(j,),
                            device_id_type=pl.DeviceIdType.MESH,
                        )
        if _CREDIT:
            for j in range(N_DEV):
                @pl.when(j < my_pos)
                def _():
                    pl.semaphore_signal(
                        ready_sems.at[my_pos], inc=1,
                        device_id=(j,),
                        device_id_type=pl.DeviceIdType.MESH,
                    )

        copy_in = pltpu.make_async_copy(x_hbm, xv_ref, in_sem)
        copy_in.start()
        row = lax.broadcasted_iota(jnp.int32, (B, B), 0)
        col = lax.broadcasted_iota(jnp.int32, (B, B), 1)
        tri = (row >= col).astype(jnp.bfloat16)
        copy_in.wait()

        xv = xv_ref[:, :]
        tot_ref[0, :] = jnp.sum(xv, axis=0)

        sends = []
        if not _NOCOMM:
            if not _AUTOBAR:
                pl.semaphore_wait(
                    barrier_sem,
                    1 if (_NOBAR or _CREDIT) else N_DEV - 1)

            for j in range(N_DEV):
                rdma = pltpu.make_async_remote_copy(
                    src_ref=tot_ref,
                    dst_ref=comm_ref.at[pl.ds(my_pos, 1)],
                    send_sem=send_sems.at[j],
                    recv_sem=recv_sems.at[my_pos],
                    device_id=(j,),
                    device_id_type=pl.DeviceIdType.MESH,
                )
                sends.append(rdma)

                @pl.when(j > my_pos)
                def _():
                    if _CREDIT:
                        pl.semaphore_wait(ready_sems.at[j], 1)
                    rdma.start()

        xb = xv.astype(jnp.bfloat16)
        cums = []
        running = jnp.zeros((1, n), jnp.float32)
        for g in range(G):
            cum = jax.lax.dot(
                tri, xb[g * B:(g + 1) * B, :],
                preferred_element_type=jnp.float32,
            ) + running
            running = cum[B - 1:B, :]
            cums.append(cum)

        acc = jnp.zeros((1, n), jnp.float32)
        if not _NOCOMM:
            for j in range(N_DEV):
                recv = pltpu.make_async_remote_copy(
                    src_ref=tot_ref,
                    dst_ref=comm_ref.at[pl.ds(j, 1)],
                    send_sem=send_sems.at[j],
                    recv_sem=recv_sems.at[j],
                    device_id=(j,),
                    device_id_type=pl.DeviceIdType.MESH,
                )

                @pl.when(j < my_pos)
                def _():
                    recv.wait_recv()
            sender = lax.broadcasted_iota(jnp.int32, (N_DEV, n), 0)
            acc = jnp.sum(
                jnp.where(sender < my_pos, comm_ref[:, :], 0.0),
                axis=0, keepdims=True)

        out_copies = []
        for g in range(G):
            stage_ref[pl.ds(g * B, B), :] = (cums[g] + acc).astype(
                jnp.bfloat16)
            cp = pltpu.make_async_copy(
                stage_ref.at[pl.ds(g * B, B)],
                out_hbm.at[pl.ds(g * B, B)],
                out_sems.at[g],
            )
            cp.start()
            out_copies.append(cp)
        for cp in out_copies:
            cp.wait()

        if not _NOCOMM:
            for j in range(N_DEV):
                @pl.when(j > my_pos)
                def _(j=j):
                    sends[j].wait_send()

    out_shape = jax.ShapeDtypeStruct((m, n), jnp.bfloat16)
    return pl.pallas_call(
        body,
        out_shape=out_shape,
        in_specs=[pl.BlockSpec(memory_space=pl.ANY)],
        out_specs=pl.BlockSpec(memory_space=pl.ANY),
        scratch_shapes=[
            pltpu.VMEM((m, n), jnp.float32),
            pltpu.VMEM((m, n), jnp.bfloat16),
            pltpu.VMEM((1, n), jnp.float32),
            pltpu.VMEM((N_DEV, n), jnp.float32),
            pltpu.SemaphoreType.DMA((N_DEV,)),
            pltpu.SemaphoreType.DMA((N_DEV,)),
            pltpu.SemaphoreType.DMA,
            pltpu.SemaphoreType.DMA((G,)),
            pltpu.SemaphoreType.REGULAR((N_DEV,)),
        ],
        compiler_params=pltpu.CompilerParams(
            collective_id=None if (_NOCOMM or _AUTOBAR) else 0),
    )(x)


# device time: 10511 ns/iter; 1.0822x vs baseline; 1.0822x over previous
import jax
import jax.numpy as jnp
from jax import lax
from jax.experimental import pallas as pl
from jax.experimental.pallas import tpu as pltpu

N_DEV = 16


def kernel(x):
    m, n = x.shape
    B = 128
    G = m // B

    def body(x_hbm, out_hbm, xv_ref, stage_ref, tot_ref, comm_ref,
             send_sems, recv_sems, in_sem, out_sems):
        my_pos = lax.axis_index("i")

        barrier_sem = pltpu.get_barrier_semaphore()
        for j in range(N_DEV):
            @pl.when(j != my_pos)
            def _():
                pl.semaphore_signal(
                    barrier_sem, inc=1,
                    device_id=(j,),
                    device_id_type=pl.DeviceIdType.MESH,
                )

        copy_in = pltpu.make_async_copy(x_hbm, xv_ref, in_sem)
        copy_in.start()
        row = lax.broadcasted_iota(jnp.int32, (B, B), 0)
        col = lax.broadcasted_iota(jnp.int32, (B, B), 1)
        tri = (row >= col).astype(jnp.bfloat16)
        copy_in.wait()

        xv = xv_ref[:, :]
        tot_ref[0, :] = jnp.sum(xv, axis=0)

        pl.semaphore_wait(barrier_sem, N_DEV - 1)

        sends = []
        for j in range(N_DEV):
            rdma = pltpu.make_async_remote_copy(
                src_ref=tot_ref,
                dst_ref=comm_ref.at[pl.ds(my_pos, 1)],
                send_sem=send_sems.at[j],
                recv_sem=recv_sems.at[my_pos],
                device_id=(j,),
                device_id_type=pl.DeviceIdType.MESH,
            )
            sends.append(rdma)

            @pl.when(j > my_pos)
            def _():
                rdma.start()

        xb = xv.astype(jnp.bfloat16)
        cums = []
        running = jnp.zeros((1, n), jnp.float32)
        for g in range(G):
            cum = jax.lax.dot(
                tri, xb[g * B:(g + 1) * B, :],
                preferred_element_type=jnp.float32,
            ) + running
            running = cum[B - 1:B, :]
            cums.append(cum)

        for j in range(N_DEV):
            recv = pltpu.make_async_remote_copy(
                src_ref=tot_ref,
                dst_ref=comm_ref.at[pl.ds(j, 1)],
                send_sem=send_sems.at[j],
                recv_sem=recv_sems.at[j],
                device_id=(j,),
                device_id_type=pl.DeviceIdType.MESH,
            )

            @pl.when(j < my_pos)
            def _():
                recv.wait_recv()

        sender = lax.broadcasted_iota(jnp.int32, (N_DEV, n), 0)
        acc = jnp.sum(
            jnp.where(sender < my_pos, comm_ref[:, :], 0.0),
            axis=0, keepdims=True)

        out_copies = []
        for g in range(G):
            stage_ref[pl.ds(g * B, B), :] = cums[g] + acc
            cp = pltpu.make_async_copy(
                stage_ref.at[pl.ds(g * B, B)],
                out_hbm.at[pl.ds(g * B, B)],
                out_sems.at[g],
            )
            cp.start()
            out_copies.append(cp)
        for cp in out_copies:
            cp.wait()

        for j in range(N_DEV):
            @pl.when(j > my_pos)
            def _(j=j):
                sends[j].wait_send()

    out_shape = jax.ShapeDtypeStruct((m, n), jnp.float32)
    return pl.pallas_call(
        body,
        out_shape=out_shape,
        in_specs=[pl.BlockSpec(memory_space=pl.ANY)],
        out_specs=pl.BlockSpec(memory_space=pl.ANY),
        scratch_shapes=[
            pltpu.VMEM((m, n), jnp.float32),
            pltpu.VMEM((m, n), jnp.float32),
            pltpu.VMEM((1, n), jnp.float32),
            pltpu.VMEM((N_DEV, n), jnp.float32),
            pltpu.SemaphoreType.DMA((N_DEV,)),
            pltpu.SemaphoreType.DMA((N_DEV,)),
            pltpu.SemaphoreType.DMA,
            pltpu.SemaphoreType.DMA((G,)),
        ],
        compiler_params=pltpu.CompilerParams(collective_id=0),
    )(x)
